# Initial kernel scaffold; baseline (speedup 1.0000x reference)
import jax
import jax.numpy as jnp
from jax.experimental import pallas as pl


def kernel(x, edge_index, edge_attr, batch, params):
    raise NotImplementedError("write your pallas kernel here")


# speedup vs baseline: 2.6410x; 2.6410x over previous
import functools

import jax
import jax.numpy as jnp
from jax import lax
from jax.experimental import pallas as pl
from jax.experimental.pallas import tpu as pltpu
from jax.experimental.pallas import tpu_sc as plsc

N = 10000
E = 320000
D = 128
DE = 16
H = 128
G = 64

NC = 2
NS = 16
NW = NC * NS
EPT = E // NW
C = 80
NCHUNK = EPT // C
NPAD = 10240
RPT = NPAD // NS
ZR = 128


def _d1(a, b):
    return jnp.dot(a.astype(jnp.bfloat16), b.astype(jnp.bfloat16),
                   preferred_element_type=jnp.float32)


BN = 1000


def _table_body(x, ws, bs, wmt, out):
    t = _d1(x[...], ws[...]) + bs[...]
    out[...] = _d1(t, wmt[...])


def _table(x, ws, bs, wmt):
    return pl.pallas_call(
        _table_body,
        grid=(N // BN,),
        in_specs=[
            pl.BlockSpec((BN, D), lambda i: (i, 0)),
            pl.BlockSpec((D, H), lambda i: (0, 0)),
            pl.BlockSpec((1, H), lambda i: (0, 0)),
            pl.BlockSpec((H, H), lambda i: (0, 0)),
        ],
        out_specs=pl.BlockSpec((BN, H), lambda i: (i, 0)),
        out_shape=jax.ShapeDtypeStruct((N, H), jnp.float32),
    )(x, ws, bs, wmt)


BE = 2000


def _epart_body(ea, we, be, wmb, bm, out):
    e1 = _d1(ea[...], we[...]) + be[...]
    out[...] = _d1(e1, wmb[...]) + bm[...]


def _epart(edge_attr, we, be, wmb, bm):
    return pl.pallas_call(
        _epart_body,
        grid=(E // BE,),
        in_specs=[
            pl.BlockSpec((BE, DE), lambda i: (i, 0)),
            pl.BlockSpec((DE, H), lambda i: (0, 0)),
            pl.BlockSpec((1, H), lambda i: (0, 0)),
            pl.BlockSpec((H, H), lambda i: (0, 0)),
            pl.BlockSpec((1, H), lambda i: (0, 0)),
        ],
        out_specs=pl.BlockSpec((BE, H), lambda i: (i, 0)),
        out_shape=jax.ShapeDtypeStruct((E, H), jnp.float32),
    )(edge_attr, we, be, wmb, bm)


def _sc_aggr_body(tbl_hbm, ep_hbm, src_hbm, dst_hbm, out_hbm,
                  src_v, dst_v, gbuf, ebuf, zbuf, aggr_sh, gsem):
    cid = lax.axis_index("c")
    sid = lax.axis_index("s")
    wid = sid * NC + cid

    zero = jnp.zeros((16,), jnp.float32)

    def zrow(r, _):
        for k in range(8):
            zbuf[r, pl.ds(k * 16, 16)] = zero
        return 0

    lax.fori_loop(0, ZR, zrow, 0)
    for j in range(RPT // ZR):
        pltpu.sync_copy(zbuf, aggr_sh.at[pl.ds(sid * RPT + j * ZR, ZR)])
    plsc.subcore_barrier()

    def chunk(j, _):
        base = wid * EPT + j * C
        pltpu.sync_copy(src_hbm.at[pl.ds(base, C)], src_v)
        pltpu.sync_copy(dst_hbm.at[pl.ds(base, C)], dst_v)
        pltpu.sync_copy(ep_hbm.at[pl.ds(base, C)], ebuf)
        pltpu.async_copy(tbl_hbm.at[src_v], gbuf, gsem).wait()

        def row(r, _):
            for k in range(8):
                sl = pl.ds(k * 16, 16)
                ebuf[r, sl] = jnp.maximum(gbuf[r, sl] + ebuf[r, sl], 0.0)
            return 0

        lax.fori_loop(0, C, row, 0)
        pltpu.sync_copy(ebuf, aggr_sh.at[dst_v], add=True)
        return 0

    lax.fori_loop(0, NCHUNK, chunk, 0)
    plsc.subcore_barrier()

    for j in range(RPT // ZR):
        r0 = sid * RPT + j * ZR
        pltpu.sync_copy(aggr_sh.at[pl.ds(r0, ZR)], out_hbm.at[cid, pl.ds(r0, ZR)])


@functools.cache
def _sc_aggr_kernel():
    return pl.kernel(
        _sc_aggr_body,
        out_type=jax.ShapeDtypeStruct((NC, NPAD, H), jnp.float32),
        mesh=plsc.VectorSubcoreMesh(core_axis_name="c", subcore_axis_name="s",
                                    num_cores=NC, num_subcores=NS),
        scratch_types=[
            pltpu.VMEM((C,), jnp.int32),
            pltpu.VMEM((C,), jnp.int32),
            pltpu.VMEM((C, H), jnp.float32),
            pltpu.VMEM((C, H), jnp.float32),
            pltpu.VMEM((ZR, H), jnp.float32),
            pltpu.VMEM_SHARED((NPAD, H), jnp.float32),
            pltpu.SemaphoreType.DMA,
        ],
    )


def _sc_aggr(tbl, ep, src, dst):
    return _sc_aggr_kernel()(tbl, ep, src, dst)


def _update_body(x, agg, wux, bux, wut, wub, bu, ws2, bs2, wmt2,
                 xn_ref, tn_ref):
    old = _d1(x[...], wux[...]) + bux[...]
    s = agg[0] + agg[1]
    xn = _d1(old, wut[...]) + _d1(s, wub[...]) + bu[...]
    xn = jnp.maximum(xn, 0.0)
    xn_ref[...] = xn
    t2 = _d1(xn, ws2[...]) + bs2[...]
    tn_ref[...] = _d1(t2, wmt2[...])


def _update(x, agg, wux, bux, wut, wub, bu, ws2, bs2, wmt2):
    return pl.pallas_call(
        _update_body,
        grid=(N // BN,),
        in_specs=[
            pl.BlockSpec((BN, D), lambda i: (i, 0)),
            pl.BlockSpec((NC, BN, H), lambda i: (0, i, 0)),
            pl.BlockSpec((D, H), lambda i: (0, 0)),
            pl.BlockSpec((1, H), lambda i: (0, 0)),
            pl.BlockSpec((H, H), lambda i: (0, 0)),
            pl.BlockSpec((H, H), lambda i: (0, 0)),
            pl.BlockSpec((1, H), lambda i: (0, 0)),
            pl.BlockSpec((D, H), lambda i: (0, 0)),
            pl.BlockSpec((1, H), lambda i: (0, 0)),
            pl.BlockSpec((H, H), lambda i: (0, 0)),
        ],
        out_specs=[
            pl.BlockSpec((BN, H), lambda i: (i, 0)),
            pl.BlockSpec((BN, H), lambda i: (i, 0)),
        ],
        out_shape=[
            jax.ShapeDtypeStruct((N, H), jnp.float32),
            jax.ShapeDtypeStruct((N, H), jnp.float32),
        ],
    )(x, agg, wux, bux, wut, wub, bu, ws2, bs2, wmt2)


def _final_body(x, agg, wux, bux, wut, wub, bu, batch, w1, b1, w2, b2,
                out_ref, sums_ref, cnt_ref):
    i = pl.program_id(0)

    @pl.when(i == 0)
    def _():
        sums_ref[...] = jnp.zeros_like(sums_ref)
        cnt_ref[...] = jnp.zeros_like(cnt_ref)

    old = _d1(x[...], wux[...]) + bux[...]
    s = agg[0] + agg[1]
    xn = _d1(old, wut[...]) + _d1(s, wub[...]) + bu[...]
    xn = jnp.maximum(xn, 0.0)

    onehot = (batch[...] == lax.broadcasted_iota(jnp.int32, (1, H), 1)
              ).astype(jnp.float32)
    dn = (((0,), (0,)), ((), ()))
    sums_ref[...] += lax.dot_general(onehot, xn, dn,
                                     preferred_element_type=jnp.float32,
                                     precision=lax.Precision.HIGHEST)
    cnt_ref[...] += lax.dot_general(onehot, jnp.ones_like(xn), dn,
                                    preferred_element_type=jnp.float32,
                                    precision=lax.Precision.HIGHEST)

    @pl.when(i == pl.num_programs(0) - 1)
    def _():
        pooled = sums_ref[:G] / jnp.maximum(cnt_ref[:G], 1.0)
        h2 = jnp.maximum(_d1(pooled, w1[...]) + b1[...], 0.0)
        out_ref[...] = _d1(h2, w2[...]) + b2[...]


def _final(x, agg, wux, bux, wut, wub, bu, batch2d, w1p, b1p, w2p, b2s):
    return pl.pallas_call(
        _final_body,
        grid=(N // BN,),
        in_specs=[
            pl.BlockSpec((BN, D), lambda i: (i, 0)),
            pl.BlockSpec((NC, BN, H), lambda i: (0, i, 0)),
            pl.BlockSpec((D, H), lambda i: (0, 0)),
            pl.BlockSpec((1, H), lambda i: (0, 0)),
            pl.BlockSpec((H, H), lambda i: (0, 0)),
            pl.BlockSpec((H, H), lambda i: (0, 0)),
            pl.BlockSpec((1, H), lambda i: (0, 0)),
            pl.BlockSpec((BN, 1), lambda i: (i, 0)),
            pl.BlockSpec((H, H), lambda i: (0, 0)),
            pl.BlockSpec((1, H), lambda i: (0, 0)),
            pl.BlockSpec((H, H), lambda i: (0, 0)),
            pl.BlockSpec((1, H), lambda i: (0, 0)),
        ],
        out_specs=[
            pl.BlockSpec((G, H), lambda i: (0, 0)),
            pl.BlockSpec((H, H), lambda i: (0, 0)),
            pl.BlockSpec((H, H), lambda i: (0, 0)),
        ],
        out_shape=[
            jax.ShapeDtypeStruct((G, H), jnp.float32),
            jax.ShapeDtypeStruct((H, H), jnp.float32),
            jax.ShapeDtypeStruct((H, H), jnp.float32),
        ],
    )(x, agg, wux, bux, wut, wub, bu, batch2d, w1p, b1p, w2p, b2s)[0]


@jax.jit
def _run(x, edge_index, edge_attr, batch, params):
    src = edge_index[0]
    dst = edge_index[1]

    def parts(conv):
        wm = conv["msg"]["W"]
        wu = conv["upd"]["W"]
        return dict(
            ws=conv["sender"]["W"], bs=conv["sender"]["b"].reshape(1, H),
            we=conv["edge"]["W"], be=conv["edge"]["b"].reshape(1, H),
            wmt=wm[:H], wmb=wm[H:], bm=conv["msg"]["b"].reshape(1, H),
            wux=conv["upd_x"]["W"], bux=conv["upd_x"]["b"].reshape(1, H),
            wut=wu[:H], wub=wu[H:], bu=conv["upd"]["b"].reshape(1, H),
        )

    p1, p2, p3 = (parts(params["conv1"]), parts(params["conv2"]),
                  parts(params["conv3"]))

    ep1 = _epart(edge_attr, p1["we"], p1["be"], p1["wmb"], p1["bm"])
    ep2 = _epart(edge_attr, p2["we"], p2["be"], p2["wmb"], p2["bm"])
    ep3 = _epart(edge_attr, p3["we"], p3["be"], p3["wmb"], p3["bm"])

    t1 = _table(x, p1["ws"], p1["bs"], p1["wmt"])
    agg1 = _sc_aggr(t1, ep1, src, dst)
    x2, t2 = _update(x, agg1, p1["wux"], p1["bux"], p1["wut"], p1["wub"],
                     p1["bu"], p2["ws"], p2["bs"], p2["wmt"])
    agg2 = _sc_aggr(t2, ep2, src, dst)
    x3, t3 = _update(x2, agg2, p2["wux"], p2["bux"], p2["wut"], p2["wub"],
                     p2["bu"], p3["ws"], p3["bs"], p3["wmt"])
    agg3 = _sc_aggr(t3, ep3, src, dst)

    w1 = params["fc1"]["W"]
    w1p = jnp.zeros((H, H), jnp.float32).at[:, : H // 2].set(w1)
    b1p = jnp.zeros((1, H), jnp.float32).at[0, : H // 2].set(params["fc1"]["b"])
    w2 = params["fc2"]["W"]
    w2p = jnp.zeros((H, H), jnp.float32).at[: H // 2, :1].set(w2)
    b2s = jnp.full((1, H), params["fc2"]["b"][0], jnp.float32)

    out = _final(x3, agg3, p3["wux"], p3["bux"], p3["wut"], p3["wub"],
                 p3["bu"], batch.reshape(N, 1), w1p, b1p, w2p, b2s)
    return out[:, 0]


def kernel(x, edge_index, edge_attr, batch, params):
    return _run(x, edge_index, edge_attr, batch, params)


# speedup vs baseline: 4.2313x; 1.6022x over previous
import functools

import jax
import jax.numpy as jnp
from jax import lax
from jax.experimental import pallas as pl
from jax.experimental.pallas import tpu as pltpu
from jax.experimental.pallas import tpu_sc as plsc

N = 10000
E = 320000
D = 128
DE = 16
H = 128
G = 64

NC = 2
NS = 16
NW = NC * NS
EPT = E // NW
C = 40
NCHUNK = EPT // C
NPAD = 10240
RPT = NPAD // NS
ZR = 32


def _d1(a, b):
    return jnp.dot(a.astype(jnp.bfloat16), b.astype(jnp.bfloat16),
                   preferred_element_type=jnp.float32)


BN = 1000


def _table_body(x, ws, bs, wmt, out):
    t = _d1(x[...], ws[...]) + bs[...]
    out[...] = _d1(t, wmt[...])


def _table(x, ws, bs, wmt):
    return pl.pallas_call(
        _table_body,
        grid=(N // BN,),
        in_specs=[
            pl.BlockSpec((BN, D), lambda i: (i, 0)),
            pl.BlockSpec((D, H), lambda i: (0, 0)),
            pl.BlockSpec((1, H), lambda i: (0, 0)),
            pl.BlockSpec((H, H), lambda i: (0, 0)),
        ],
        out_specs=pl.BlockSpec((BN, H), lambda i: (i, 0)),
        out_shape=jax.ShapeDtypeStruct((N, H), jnp.float32),
    )(x, ws, bs, wmt)


BE = 2000


def _epart_body(ea, we, be, wmb, bm, out):
    e1 = _d1(ea[...], we[...]) + be[...]
    out[...] = _d1(e1, wmb[...]) + bm[...]


def _epart(edge_attr, we, be, wmb, bm):
    return pl.pallas_call(
        _epart_body,
        grid=(E // BE,),
        in_specs=[
            pl.BlockSpec((BE, DE), lambda i: (i, 0)),
            pl.BlockSpec((DE, H), lambda i: (0, 0)),
            pl.BlockSpec((1, H), lambda i: (0, 0)),
            pl.BlockSpec((H, H), lambda i: (0, 0)),
            pl.BlockSpec((1, H), lambda i: (0, 0)),
        ],
        out_specs=pl.BlockSpec((BE, H), lambda i: (i, 0)),
        out_shape=jax.ShapeDtypeStruct((E, H), jnp.float32),
    )(edge_attr, we, be, wmb, bm)


def _sc_aggr_body(tbl_hbm, ep_hbm, src_hbm, dst_hbm, out_hbm,
                  sidx, didx, dst_sc, gbuf, ebuf, zbuf, aggr_sh,
                  semi, semg, seme, semsc):
    cid = lax.axis_index("c")
    sid = lax.axis_index("s")
    wid = sid * NC + cid
    ebase = wid * EPT

    def ep_slice(j):
        return ep_hbm.at[pl.ds(ebase + j * C, C)]

    def gather_cp(j, b):
        return (tbl_hbm.at[sidx.at[pl.ds(j * C, C)]], gbuf.at[b], semg[b])

    def scatter_cp(b):
        return (ebuf.at[b], aggr_sh.at[dst_sc.at[b]], semsc[b])

    ci = pltpu.async_copy(src_hbm.at[pl.ds(ebase, EPT)], sidx, semi)
    cd = pltpu.async_copy(dst_hbm.at[pl.ds(ebase, EPT)], didx, semi)
    zero = jnp.zeros((16,), jnp.float32)

    def zrow(r, _):
        for k in range(8):
            zbuf[r, pl.ds(k * 16, 16)] = zero
        return 0

    lax.fori_loop(0, ZR, zrow, 0)
    ci.wait()
    cd.wait()

    pltpu.async_copy(ep_slice(0), ebuf.at[0], seme[0])
    pltpu.async_copy(*gather_cp(0, 0))
    for j in range(RPT // ZR):
        pltpu.sync_copy(zbuf, aggr_sh.at[pl.ds(sid * RPT + j * ZR, ZR)])
    plsc.subcore_barrier()

    def step(j, b):
        bn = 1 - b
        pltpu.make_async_copy(ep_slice(j), ebuf.at[b], seme[b]).wait()
        pltpu.make_async_copy(*gather_cp(j, b)).wait()

        @pl.when(j >= 1)
        def _():
            pltpu.make_async_copy(*scatter_cp(bn)).wait()

        @pl.when(j + 1 < NCHUNK)
        def _():
            pltpu.async_copy(ep_slice(j + 1), ebuf.at[bn], seme[bn])
            pltpu.async_copy(*gather_cp(j + 1, bn))

        @plsc.parallel_loop(0, C, 1, unroll=2)
        def _(r):
            for k in range(8):
                sl = pl.ds(k * 16, 16)
                ebuf[b, r, sl] = jnp.maximum(gbuf[b, r, sl] + ebuf[b, r, sl],
                                             0.0)

        for off in (0, 16, C - 16):
            dst_sc[b, pl.ds(off, 16)] = didx[pl.ds(j * C + off, 16)]
        pltpu.async_copy(ebuf.at[b], aggr_sh.at[dst_sc.at[b]], semsc[b],
                         add=True)

    def pair(i, _):
        step(2 * i, 0)
        step(2 * i + 1, 1)
        return 0

    lax.fori_loop(0, NCHUNK // 2, pair, 0)
    pltpu.make_async_copy(*scatter_cp(1)).wait()
    plsc.subcore_barrier()

    for j in range(RPT // ZR):
        r0 = sid * RPT + j * ZR
        pltpu.sync_copy(aggr_sh.at[pl.ds(r0, ZR)], out_hbm.at[cid, pl.ds(r0, ZR)])


@functools.cache
def _sc_aggr_kernel():
    return pl.kernel(
        _sc_aggr_body,
        out_type=jax.ShapeDtypeStruct((NC, NPAD, H), jnp.float32),
        mesh=plsc.VectorSubcoreMesh(core_axis_name="c", subcore_axis_name="s",
                                    num_cores=NC, num_subcores=NS),
        scratch_types=[
            pltpu.VMEM((EPT,), jnp.int32),
            pltpu.VMEM((EPT,), jnp.int32),
            pltpu.VMEM((2, C), jnp.int32),
            pltpu.VMEM((2, C, H), jnp.float32),
            pltpu.VMEM((2, C, H), jnp.float32),
            pltpu.VMEM((ZR, H), jnp.float32),
            pltpu.VMEM_SHARED((NPAD, H), jnp.float32),
            pltpu.SemaphoreType.DMA,
            [pltpu.SemaphoreType.DMA] * 2,
            [pltpu.SemaphoreType.DMA] * 2,
            [pltpu.SemaphoreType.DMA] * 2,
        ],
    )


def _sc_aggr(tbl, ep, src, dst):
    return _sc_aggr_kernel()(tbl, ep, src, dst)


def _update_body(x, agg, wux, bux, wut, wub, bu, ws2, bs2, wmt2,
                 xn_ref, tn_ref):
    old = _d1(x[...], wux[...]) + bux[...]
    s = agg[0] + agg[1]
    xn = _d1(old, wut[...]) + _d1(s, wub[...]) + bu[...]
    xn = jnp.maximum(xn, 0.0)
    xn_ref[...] = xn
    t2 = _d1(xn, ws2[...]) + bs2[...]
    tn_ref[...] = _d1(t2, wmt2[...])


def _update(x, agg, wux, bux, wut, wub, bu, ws2, bs2, wmt2):
    return pl.pallas_call(
        _update_body,
        grid=(N // BN,),
        in_specs=[
            pl.BlockSpec((BN, D), lambda i: (i, 0)),
            pl.BlockSpec((NC, BN, H), lambda i: (0, i, 0)),
            pl.BlockSpec((D, H), lambda i: (0, 0)),
            pl.BlockSpec((1, H), lambda i: (0, 0)),
            pl.BlockSpec((H, H), lambda i: (0, 0)),
            pl.BlockSpec((H, H), lambda i: (0, 0)),
            pl.BlockSpec((1, H), lambda i: (0, 0)),
            pl.BlockSpec((D, H), lambda i: (0, 0)),
            pl.BlockSpec((1, H), lambda i: (0, 0)),
            pl.BlockSpec((H, H), lambda i: (0, 0)),
        ],
        out_specs=[
            pl.BlockSpec((BN, H), lambda i: (i, 0)),
            pl.BlockSpec((BN, H), lambda i: (i, 0)),
        ],
        out_shape=[
            jax.ShapeDtypeStruct((N, H), jnp.float32),
            jax.ShapeDtypeStruct((N, H), jnp.float32),
        ],
    )(x, agg, wux, bux, wut, wub, bu, ws2, bs2, wmt2)


def _final_body(x, agg, wux, bux, wut, wub, bu, batch, w1, b1, w2, b2,
                out_ref, sums_ref, cnt_ref):
    i = pl.program_id(0)

    @pl.when(i == 0)
    def _():
        sums_ref[...] = jnp.zeros_like(sums_ref)
        cnt_ref[...] = jnp.zeros_like(cnt_ref)

    old = _d1(x[...], wux[...]) + bux[...]
    s = agg[0] + agg[1]
    xn = _d1(old, wut[...]) + _d1(s, wub[...]) + bu[...]
    xn = jnp.maximum(xn, 0.0)

    onehot = (batch[...] == lax.broadcasted_iota(jnp.int32, (1, H), 1)
              ).astype(jnp.float32)
    dn = (((0,), (0,)), ((), ()))
    sums_ref[...] += lax.dot_general(onehot, xn, dn,
                                     preferred_element_type=jnp.float32,
                                     precision=lax.Precision.HIGHEST)
    cnt_ref[...] += lax.dot_general(onehot, jnp.ones_like(xn), dn,
                                    preferred_element_type=jnp.float32,
                                    precision=lax.Precision.HIGHEST)

    @pl.when(i == pl.num_programs(0) - 1)
    def _():
        pooled = sums_ref[:G] / jnp.maximum(cnt_ref[:G], 1.0)
        h2 = jnp.maximum(_d1(pooled, w1[...]) + b1[...], 0.0)
        out_ref[...] = _d1(h2, w2[...]) + b2[...]


def _final(x, agg, wux, bux, wut, wub, bu, batch2d, w1p, b1p, w2p, b2s):
    return pl.pallas_call(
        _final_body,
        grid=(N // BN,),
        in_specs=[
            pl.BlockSpec((BN, D), lambda i: (i, 0)),
            pl.BlockSpec((NC, BN, H), lambda i: (0, i, 0)),
            pl.BlockSpec((D, H), lambda i: (0, 0)),
            pl.BlockSpec((1, H), lambda i: (0, 0)),
            pl.BlockSpec((H, H), lambda i: (0, 0)),
            pl.BlockSpec((H, H), lambda i: (0, 0)),
            pl.BlockSpec((1, H), lambda i: (0, 0)),
            pl.BlockSpec((BN, 1), lambda i: (i, 0)),
            pl.BlockSpec((H, H), lambda i: (0, 0)),
            pl.BlockSpec((1, H), lambda i: (0, 0)),
            pl.BlockSpec((H, H), lambda i: (0, 0)),
            pl.BlockSpec((1, H), lambda i: (0, 0)),
        ],
        out_specs=[
            pl.BlockSpec((G, H), lambda i: (0, 0)),
            pl.BlockSpec((H, H), lambda i: (0, 0)),
            pl.BlockSpec((H, H), lambda i: (0, 0)),
        ],
        out_shape=[
            jax.ShapeDtypeStruct((G, H), jnp.float32),
            jax.ShapeDtypeStruct((H, H), jnp.float32),
            jax.ShapeDtypeStruct((H, H), jnp.float32),
        ],
    )(x, agg, wux, bux, wut, wub, bu, batch2d, w1p, b1p, w2p, b2s)[0]


@jax.jit
def _run(x, edge_index, edge_attr, batch, params):
    src = edge_index[0]
    dst = edge_index[1]

    def parts(conv):
        wm = conv["msg"]["W"]
        wu = conv["upd"]["W"]
        return dict(
            ws=conv["sender"]["W"], bs=conv["sender"]["b"].reshape(1, H),
            we=conv["edge"]["W"], be=conv["edge"]["b"].reshape(1, H),
            wmt=wm[:H], wmb=wm[H:], bm=conv["msg"]["b"].reshape(1, H),
            wux=conv["upd_x"]["W"], bux=conv["upd_x"]["b"].reshape(1, H),
            wut=wu[:H], wub=wu[H:], bu=conv["upd"]["b"].reshape(1, H),
        )

    p1, p2, p3 = (parts(params["conv1"]), parts(params["conv2"]),
                  parts(params["conv3"]))

    ep1 = _epart(edge_attr, p1["we"], p1["be"], p1["wmb"], p1["bm"])
    ep2 = _epart(edge_attr, p2["we"], p2["be"], p2["wmb"], p2["bm"])
    ep3 = _epart(edge_attr, p3["we"], p3["be"], p3["wmb"], p3["bm"])

    t1 = _table(x, p1["ws"], p1["bs"], p1["wmt"])
    agg1 = _sc_aggr(t1, ep1, src, dst)
    x2, t2 = _update(x, agg1, p1["wux"], p1["bux"], p1["wut"], p1["wub"],
                     p1["bu"], p2["ws"], p2["bs"], p2["wmt"])
    agg2 = _sc_aggr(t2, ep2, src, dst)
    x3, t3 = _update(x2, agg2, p2["wux"], p2["bux"], p2["wut"], p2["wub"],
                     p2["bu"], p3["ws"], p3["bs"], p3["wmt"])
    agg3 = _sc_aggr(t3, ep3, src, dst)

    w1 = params["fc1"]["W"]
    w1p = jnp.zeros((H, H), jnp.float32).at[:, : H // 2].set(w1)
    b1p = jnp.zeros((1, H), jnp.float32).at[0, : H // 2].set(params["fc1"]["b"])
    w2 = params["fc2"]["W"]
    w2p = jnp.zeros((H, H), jnp.float32).at[: H // 2, :1].set(w2)
    b2s = jnp.full((1, H), params["fc2"]["b"][0], jnp.float32)

    out = _final(x3, agg3, p3["wux"], p3["bux"], p3["wut"], p3["wub"],
                 p3["bu"], batch.reshape(N, 1), w1p, b1p, w2p, b2s)
    return out[:, 0]


def kernel(x, edge_index, edge_attr, batch, params):
    return _run(x, edge_index, edge_attr, batch, params)


# speedup vs baseline: 4.2418x; 1.0025x over previous
import functools

import jax
import jax.numpy as jnp
from jax import lax
from jax.experimental import pallas as pl
from jax.experimental.pallas import tpu as pltpu
from jax.experimental.pallas import tpu_sc as plsc

N = 10000
E = 320000
D = 128
DE = 16
H = 128
G = 64

NC = 2
NS = 16
NW = NC * NS
EPT = E // NW
C = 40
NCHUNK = EPT // C
NPAD = 10240
RPT = NPAD // NS
ZR = 32


def _d1(a, b):
    return jnp.dot(a.astype(jnp.bfloat16), b.astype(jnp.bfloat16),
                   preferred_element_type=jnp.float32)


BN = 1000


def _table_body(x, ws, bs, wmt, out):
    t = _d1(x[...], ws[...]) + bs[...]
    out[...] = _d1(t, wmt[...])


def _table(x, ws, bs, wmt):
    return pl.pallas_call(
        _table_body,
        grid=(N // BN,),
        in_specs=[
            pl.BlockSpec((BN, D), lambda i: (i, 0)),
            pl.BlockSpec((D, H), lambda i: (0, 0)),
            pl.BlockSpec((1, H), lambda i: (0, 0)),
            pl.BlockSpec((H, H), lambda i: (0, 0)),
        ],
        out_specs=pl.BlockSpec((BN, H), lambda i: (i, 0)),
        out_shape=jax.ShapeDtypeStruct((N, H), jnp.float32),
    )(x, ws, bs, wmt)


BE = 2000


def _epart_body(ea, we, be, wmb, bm, out):
    e1 = _d1(ea[...], we[...]) + be[...]
    out[...] = _d1(e1, wmb[...]) + bm[...]


def _epart(edge_attr, we, be, wmb, bm):
    return pl.pallas_call(
        _epart_body,
        grid=(E // BE,),
        in_specs=[
            pl.BlockSpec((BE, DE), lambda i: (i, 0)),
            pl.BlockSpec((DE, H), lambda i: (0, 0)),
            pl.BlockSpec((1, H), lambda i: (0, 0)),
            pl.BlockSpec((H, H), lambda i: (0, 0)),
            pl.BlockSpec((1, H), lambda i: (0, 0)),
        ],
        out_specs=pl.BlockSpec((BE, H), lambda i: (i, 0)),
        out_shape=jax.ShapeDtypeStruct((E, H), jnp.float32),
    )(edge_attr, we, be, wmb, bm)


def _sc_aggr_body(tbl_hbm, ep_hbm, src_hbm, dst_hbm, out_hbm,
                  sidx, didx, dst_sc, gbuf, ebuf, zbuf, aggr_sh,
                  semi, semg, seme, semsc):
    cid = lax.axis_index("c")
    sid = lax.axis_index("s")
    wid = sid * NC + cid
    ebase = wid * EPT

    def ep_slice(j):
        return ep_hbm.at[pl.ds(ebase + j * C, C)]

    def gather_cp(j, b):
        return (tbl_hbm.at[sidx.at[pl.ds(j * C, C)]], gbuf.at[b], semg[b])

    def scatter_cp(b):
        return (ebuf.at[b], aggr_sh.at[dst_sc.at[b]], semsc[b])

    ci = pltpu.async_copy(src_hbm.at[pl.ds(ebase, EPT)], sidx, semi)
    cd = pltpu.async_copy(dst_hbm.at[pl.ds(ebase, EPT)], didx, semi)
    zero = jnp.zeros((16,), jnp.float32)

    def zrow(r, _):
        for k in range(8):
            zbuf[r, pl.ds(k * 16, 16)] = zero
        return 0

    lax.fori_loop(0, ZR, zrow, 0)
    ci.wait()
    cd.wait()

    pltpu.async_copy(ep_slice(0), ebuf.at[0], seme[0])
    pltpu.async_copy(*gather_cp(0, 0))
    for j in range(RPT // ZR):
        pltpu.sync_copy(zbuf, aggr_sh.at[pl.ds(sid * RPT + j * ZR, ZR)])
    plsc.subcore_barrier()

    def step(j, b):
        bn = 1 - b
        pltpu.make_async_copy(ep_slice(j), ebuf.at[b], seme[b]).wait()
        pltpu.make_async_copy(*gather_cp(j, b)).wait()

        @pl.when(j >= 1)
        def _():
            pltpu.make_async_copy(*scatter_cp(bn)).wait()

        @pl.when(j + 1 < NCHUNK)
        def _():
            pltpu.async_copy(ep_slice(j + 1), ebuf.at[bn], seme[bn])
            pltpu.async_copy(*gather_cp(j + 1, bn))

        @plsc.parallel_loop(0, C, 1, unroll=2)
        def _(r):
            for k in range(8):
                sl = pl.ds(k * 16, 16)
                ebuf[b, r, sl] = jnp.maximum(gbuf[b, r, sl] + ebuf[b, r, sl],
                                             0.0)

        for off in (0, 16, C - 16):
            dst_sc[b, pl.ds(off, 16)] = didx[pl.ds(j * C + off, 16)]
        pltpu.async_copy(ebuf.at[b], aggr_sh.at[dst_sc.at[b]], semsc[b],
                         add=True)

    def pair(i, _):
        step(2 * i, 0)
        step(2 * i + 1, 1)
        return 0

    lax.fori_loop(0, NCHUNK // 2, pair, 0)
    pltpu.make_async_copy(*scatter_cp(1)).wait()
    plsc.subcore_barrier()

    for j in range(RPT // ZR):
        r0 = sid * RPT + j * ZR
        pltpu.sync_copy(aggr_sh.at[pl.ds(r0, ZR)], out_hbm.at[cid, pl.ds(r0, ZR)])


@functools.cache
def _sc_aggr_kernel():
    return pl.kernel(
        _sc_aggr_body,
        out_type=jax.ShapeDtypeStruct((NC, NPAD, H), jnp.float32),
        mesh=plsc.VectorSubcoreMesh(core_axis_name="c", subcore_axis_name="s",
                                    num_cores=NC, num_subcores=NS),
        scratch_types=[
            pltpu.VMEM((EPT,), jnp.int32),
            pltpu.VMEM((EPT,), jnp.int32),
            pltpu.VMEM((2, C), jnp.int32),
            pltpu.VMEM((2, C, H), jnp.float32),
            pltpu.VMEM((2, C, H), jnp.float32),
            pltpu.VMEM((ZR, H), jnp.float32),
            pltpu.VMEM_SHARED((NPAD, H), jnp.float32),
            pltpu.SemaphoreType.DMA,
            [pltpu.SemaphoreType.DMA] * 2,
            [pltpu.SemaphoreType.DMA] * 2,
            [pltpu.SemaphoreType.DMA] * 2,
        ],
    )


def _sc_aggr(tbl, ep, src, dst):
    return _sc_aggr_kernel()(tbl, ep, src, dst)


def _update_body(x, agg, wux, bux, wu, bu, ws2, bs2, wmt2,
                 xn_ref, tn_ref):
    old = _d1(x[...], wux[...]) + bux[...]
    s = agg[0] + agg[1]
    xn = _d1(jnp.concatenate([old, s], axis=-1), wu[...]) + bu[...]
    xn = jnp.maximum(xn, 0.0)
    xn_ref[...] = xn
    t2 = _d1(xn, ws2[...]) + bs2[...]
    tn_ref[...] = _d1(t2, wmt2[...])


def _update(x, agg, wux, bux, wu, bu, ws2, bs2, wmt2):
    return pl.pallas_call(
        _update_body,
        grid=(N // BN,),
        in_specs=[
            pl.BlockSpec((BN, D), lambda i: (i, 0)),
            pl.BlockSpec((NC, BN, H), lambda i: (0, i, 0)),
            pl.BlockSpec((D, H), lambda i: (0, 0)),
            pl.BlockSpec((1, H), lambda i: (0, 0)),
            pl.BlockSpec((2 * H, H), lambda i: (0, 0)),
            pl.BlockSpec((1, H), lambda i: (0, 0)),
            pl.BlockSpec((D, H), lambda i: (0, 0)),
            pl.BlockSpec((1, H), lambda i: (0, 0)),
            pl.BlockSpec((H, H), lambda i: (0, 0)),
        ],
        out_specs=[
            pl.BlockSpec((BN, H), lambda i: (i, 0)),
            pl.BlockSpec((BN, H), lambda i: (i, 0)),
        ],
        out_shape=[
            jax.ShapeDtypeStruct((N, H), jnp.float32),
            jax.ShapeDtypeStruct((N, H), jnp.float32),
        ],
    )(x, agg, wux, bux, wu, bu, ws2, bs2, wmt2)


def _final_body(x, agg, wux, bux, wu, bu, batch, w1, b1, w2, b2,
                out_ref, sums_ref, cnt_ref):
    i = pl.program_id(0)

    @pl.when(i == 0)
    def _():
        sums_ref[...] = jnp.zeros_like(sums_ref)
        cnt_ref[...] = jnp.zeros_like(cnt_ref)

    old = _d1(x[...], wux[...]) + bux[...]
    s = agg[0] + agg[1]
    xn = _d1(jnp.concatenate([old, s], axis=-1), wu[...]) + bu[...]
    xn = jnp.maximum(xn, 0.0)

    onehot = (batch[...] == lax.broadcasted_iota(jnp.int32, (1, H), 1)
              ).astype(jnp.float32)
    dn = (((0,), (0,)), ((), ()))
    sums_ref[...] += lax.dot_general(onehot, xn, dn,
                                     preferred_element_type=jnp.float32,
                                     precision=lax.Precision.HIGHEST)
    cnt_ref[...] += lax.dot_general(onehot, jnp.ones_like(xn), dn,
                                    preferred_element_type=jnp.float32,
                                    precision=lax.Precision.HIGHEST)

    @pl.when(i == pl.num_programs(0) - 1)
    def _():
        pooled = sums_ref[:G] / jnp.maximum(cnt_ref[:G], 1.0)
        h2 = jnp.maximum(_d1(pooled, w1[...]) + b1[...], 0.0)
        out_ref[...] = _d1(h2, w2[...]) + b2[...]


def _final(x, agg, wux, bux, wu, bu, batch2d, w1p, b1p, w2p, b2s):
    return pl.pallas_call(
        _final_body,
        grid=(N // BN,),
        in_specs=[
            pl.BlockSpec((BN, D), lambda i: (i, 0)),
            pl.BlockSpec((NC, BN, H), lambda i: (0, i, 0)),
            pl.BlockSpec((D, H), lambda i: (0, 0)),
            pl.BlockSpec((1, H), lambda i: (0, 0)),
            pl.BlockSpec((2 * H, H), lambda i: (0, 0)),
            pl.BlockSpec((1, H), lambda i: (0, 0)),
            pl.BlockSpec((BN, 1), lambda i: (i, 0)),
            pl.BlockSpec((H, H), lambda i: (0, 0)),
            pl.BlockSpec((1, H), lambda i: (0, 0)),
            pl.BlockSpec((H, H), lambda i: (0, 0)),
            pl.BlockSpec((1, H), lambda i: (0, 0)),
        ],
        out_specs=[
            pl.BlockSpec((G, H), lambda i: (0, 0)),
            pl.BlockSpec((H, H), lambda i: (0, 0)),
            pl.BlockSpec((H, H), lambda i: (0, 0)),
        ],
        out_shape=[
            jax.ShapeDtypeStruct((G, H), jnp.float32),
            jax.ShapeDtypeStruct((H, H), jnp.float32),
            jax.ShapeDtypeStruct((H, H), jnp.float32),
        ],
    )(x, agg, wux, bux, wu, bu, batch2d, w1p, b1p, w2p, b2s)[0]


@jax.jit
def _run(x, edge_index, edge_attr, batch, params):
    src = edge_index[0]
    dst = edge_index[1]

    def parts(conv):
        wm = conv["msg"]["W"]
        wu = conv["upd"]["W"]
        return dict(
            ws=conv["sender"]["W"], bs=conv["sender"]["b"].reshape(1, H),
            we=conv["edge"]["W"], be=conv["edge"]["b"].reshape(1, H),
            wmt=wm[:H], wmb=wm[H:], bm=conv["msg"]["b"].reshape(1, H),
            wux=conv["upd_x"]["W"], bux=conv["upd_x"]["b"].reshape(1, H),
            wu=wu, bu=conv["upd"]["b"].reshape(1, H),
        )

    p1, p2, p3 = (parts(params["conv1"]), parts(params["conv2"]),
                  parts(params["conv3"]))

    ep1 = _epart(edge_attr, p1["we"], p1["be"], p1["wmb"], p1["bm"])
    ep2 = _epart(edge_attr, p2["we"], p2["be"], p2["wmb"], p2["bm"])
    ep3 = _epart(edge_attr, p3["we"], p3["be"], p3["wmb"], p3["bm"])

    t1 = _table(x, p1["ws"], p1["bs"], p1["wmt"])
    agg1 = _sc_aggr(t1, ep1, src, dst)
    x2, t2 = _update(x, agg1, p1["wux"], p1["bux"], p1["wu"],
                     p1["bu"], p2["ws"], p2["bs"], p2["wmt"])
    agg2 = _sc_aggr(t2, ep2, src, dst)
    x3, t3 = _update(x2, agg2, p2["wux"], p2["bux"], p2["wu"],
                     p2["bu"], p3["ws"], p3["bs"], p3["wmt"])
    agg3 = _sc_aggr(t3, ep3, src, dst)

    w1 = params["fc1"]["W"]
    w1p = jnp.zeros((H, H), jnp.float32).at[:, : H // 2].set(w1)
    b1p = jnp.zeros((1, H), jnp.float32).at[0, : H // 2].set(params["fc1"]["b"])
    w2 = params["fc2"]["W"]
    w2p = jnp.zeros((H, H), jnp.float32).at[: H // 2, :1].set(w2)
    b2s = jnp.full((1, H), params["fc2"]["b"][0], jnp.float32)

    out = _final(x3, agg3, p3["wux"], p3["bux"], p3["wu"],
                 p3["bu"], batch.reshape(N, 1), w1p, b1p, w2p, b2s)
    return out[:, 0]


def kernel(x, edge_index, edge_attr, batch, params):
    return _run(x, edge_index, edge_attr, batch, params)
